# Initial kernel scaffold; baseline (speedup 1.0000x reference)
import jax
import jax.numpy as jnp
from jax.experimental import pallas as pl
from jax.experimental.pallas import tpu as pltpu
from jax.experimental.pallas import tpu_sc as plsc

B, C, H, W = 32, 384, 32, 32
HW = H * W
N_EMBED, EMBED_DIM = 8192, 64
CODE_CHUNK = 1024
N_CHUNKS = N_EMBED // CODE_CHUNK
_PREC = jax.lax.Precision.HIGHEST


def _argmin_kernel(z_ref, w_ref, b_ref, emb_ref, ze_ref, ind_ref):
    x = z_ref[0]
    w = w_ref[...]
    ze = jnp.dot(w, x, preferred_element_type=jnp.float32,
                 precision=_PREC) + b_ref[...]
    ze_ref[0] = ze
    xn = jnp.sum(ze * ze, axis=0)
    best_val = jnp.full((HW,), jnp.inf, dtype=jnp.float32)
    best_idx = jnp.zeros((HW,), dtype=jnp.int32)
    for c in range(N_CHUNKS):
        e = emb_ref[pl.ds(c * CODE_CHUNK, CODE_CHUNK), :]
        en = jnp.sum(e * e, axis=1)
        s = jnp.dot(e, ze, preferred_element_type=jnp.float32,
                    precision=_PREC)
        dmat = (xn[None, :] - 2.0 * s) + en[:, None]
        m = jnp.min(dmat, axis=0)
        iota = jax.lax.broadcasted_iota(jnp.int32, (CODE_CHUNK, HW), 0)
        cand = jnp.where(dmat == m[None, :], iota + c * CODE_CHUNK,
                         jnp.int32(2**30))
        ci = jnp.min(cand, axis=0)
        upd = m < best_val
        best_val = jnp.where(upd, m, best_val)
        best_idx = jnp.where(upd, ci, best_idx)
    ind_ref[0, 0] = best_idx


def _project_and_argmin(z3, proj_w, proj_b2, embed):
    return pl.pallas_call(
        _argmin_kernel,
        grid=(B,),
        in_specs=[
            pl.BlockSpec((1, C, HW), lambda b: (b, 0, 0)),
            pl.BlockSpec((EMBED_DIM, C), lambda b: (0, 0)),
            pl.BlockSpec((EMBED_DIM, 1), lambda b: (0, 0)),
            pl.BlockSpec((N_EMBED, EMBED_DIM), lambda b: (0, 0)),
        ],
        out_specs=[
            pl.BlockSpec((1, EMBED_DIM, HW), lambda b: (b, 0, 0)),
            pl.BlockSpec((1, 1, HW), lambda b: (b, 0, 0)),
        ],
        out_shape=[
            jax.ShapeDtypeStruct((B, EMBED_DIM, HW), jnp.float32),
            jax.ShapeDtypeStruct((B, 1, HW), jnp.int32),
        ],
    )(z3, proj_w, proj_b2, embed)


_GATHER_WIN = 128
_N_IDX = B * HW


def _sc_gather(embed, ind_flat):
    mesh = plsc.VectorSubcoreMesh(core_axis_name="core",
                                  subcore_axis_name="subcore")

    @pl.kernel(out_type=jax.ShapeDtypeStruct((_N_IDX, EMBED_DIM), jnp.float32),
               mesh=mesh)
    def gather_kernel(x_hbm, i_hbm, o_hbm):
        def body(i_vmem, o_vmem):
            pltpu.sync_copy(x_hbm.at[i_vmem.at[0]], o_vmem)

        pltpu.emit_pipeline(
            body,
            grid=(_N_IDX // _GATHER_WIN,),
            in_specs=[pl.BlockSpec((1, _GATHER_WIN), lambda i: (0, i))],
            out_specs=[pl.BlockSpec((_GATHER_WIN, EMBED_DIM),
                                    lambda i: (i, 0))],
            core_axis_name=("core", "subcore"),
            dimension_semantics=(pltpu.PARALLEL,),
        )(i_hbm, o_hbm)

    return gather_kernel(embed, ind_flat)


def _finish_kernel(zq_ref, ze_ref, out_ref, loss_ref):
    zq = jnp.transpose(zq_ref[0], (1, 0))
    ze = ze_ref[0]
    diff = zq - ze
    out_ref[0] = ze + diff

    @pl.when(pl.program_id(0) == 0)
    def _():
        loss_ref[0, 0] = 0.0

    loss_ref[0, 0] += jnp.sum(diff * diff)


def _finish(zq_rows3, ze_t):
    return pl.pallas_call(
        _finish_kernel,
        grid=(B,),
        in_specs=[
            pl.BlockSpec((1, HW, EMBED_DIM), lambda b: (b, 0, 0)),
            pl.BlockSpec((1, EMBED_DIM, HW), lambda b: (b, 0, 0)),
        ],
        out_specs=[
            pl.BlockSpec((1, EMBED_DIM, HW), lambda b: (b, 0, 0)),
            pl.BlockSpec((1, 1), lambda b: (0, 0)),
        ],
        out_shape=[
            jax.ShapeDtypeStruct((B, EMBED_DIM, HW), jnp.float32),
            jax.ShapeDtypeStruct((1, 1), jnp.float32),
        ],
    )(zq_rows3, ze_t)


def kernel(z, wait_to_init, proj_w, proj_b, embed):
    z3 = z.reshape(B, C, HW)
    proj_b2 = proj_b.reshape(EMBED_DIM, 1)
    ze_t, ind3 = _project_and_argmin(z3, proj_w, proj_b2, embed)
    ind_flat = ind3.reshape(1, _N_IDX)
    zq_rows = _sc_gather(embed, ind_flat)
    zq_out, loss_sum = _finish(zq_rows.reshape(B, HW, EMBED_DIM), ze_t)
    m = loss_sum[0, 0] / jnp.float32(_N_IDX * EMBED_DIM)
    latent_loss = (0.25 * m + m) * 10.0
    z_q = zq_out.reshape(B, EMBED_DIM, H, W)
    ind = ind3.reshape(B, H, W)
    return (z_q, latent_loss, ind)


# speedup vs baseline: 1.2826x; 1.2826x over previous
import jax
import jax.numpy as jnp
from jax.experimental import pallas as pl
from jax.experimental.pallas import tpu as pltpu
from jax.experimental.pallas import tpu_sc as plsc

B, C, H, W = 32, 384, 32, 32
HW = H * W
N_EMBED, EMBED_DIM = 8192, 64
CODE_CHUNK = 1024
N_CHUNKS = N_EMBED // CODE_CHUNK
_PREC = jax.lax.Precision.DEFAULT


def _argmin_kernel(z_ref, w_ref, b_ref, emb_ref, ze_ref, ind_ref):
    x = z_ref[0]
    w = w_ref[...]
    ze = jnp.dot(w, x, preferred_element_type=jnp.float32,
                 precision=_PREC) + b_ref[...]
    ze_ref[0] = ze
    xn = jnp.sum(ze * ze, axis=0)
    best_val = jnp.full((HW,), jnp.inf, dtype=jnp.float32)
    best_idx = jnp.zeros((HW,), dtype=jnp.int32)
    for c in range(N_CHUNKS):
        e = emb_ref[pl.ds(c * CODE_CHUNK, CODE_CHUNK), :]
        en = jnp.sum(e * e, axis=1)
        s = jnp.dot(e, ze, preferred_element_type=jnp.float32,
                    precision=_PREC)
        dmat = (xn[None, :] - 2.0 * s) + en[:, None]
        m = jnp.min(dmat, axis=0)
        iota = jax.lax.broadcasted_iota(jnp.int32, (CODE_CHUNK, HW), 0)
        cand = jnp.where(dmat == m[None, :], iota + c * CODE_CHUNK,
                         jnp.int32(2**30))
        ci = jnp.min(cand, axis=0)
        upd = m < best_val
        best_val = jnp.where(upd, m, best_val)
        best_idx = jnp.where(upd, ci, best_idx)
    ind_ref[0, 0] = best_idx


def _project_and_argmin(z3, proj_w, proj_b2, embed):
    return pl.pallas_call(
        _argmin_kernel,
        grid=(B,),
        in_specs=[
            pl.BlockSpec((1, C, HW), lambda b: (b, 0, 0)),
            pl.BlockSpec((EMBED_DIM, C), lambda b: (0, 0)),
            pl.BlockSpec((EMBED_DIM, 1), lambda b: (0, 0)),
            pl.BlockSpec((N_EMBED, EMBED_DIM), lambda b: (0, 0)),
        ],
        out_specs=[
            pl.BlockSpec((1, EMBED_DIM, HW), lambda b: (b, 0, 0)),
            pl.BlockSpec((1, 1, HW), lambda b: (b, 0, 0)),
        ],
        out_shape=[
            jax.ShapeDtypeStruct((B, EMBED_DIM, HW), jnp.float32),
            jax.ShapeDtypeStruct((B, 1, HW), jnp.int32),
        ],
    )(z3, proj_w, proj_b2, embed)


_GATHER_WIN = 128
_GATHER_DIM = 128
_N_IDX = B * HW


def _sc_gather(embed_pad, ind_flat):
    mesh = plsc.VectorSubcoreMesh(core_axis_name="core",
                                  subcore_axis_name="subcore")

    @pl.kernel(out_type=jax.ShapeDtypeStruct((_N_IDX, _GATHER_DIM),
                                             jnp.float32),
               mesh=mesh)
    def gather_kernel(x_hbm, i_hbm, o_hbm):
        def body(i_vmem, o_vmem):
            pltpu.sync_copy(x_hbm.at[i_vmem.at[0]], o_vmem)

        pltpu.emit_pipeline(
            body,
            grid=(_N_IDX // _GATHER_WIN,),
            in_specs=[pl.BlockSpec((1, _GATHER_WIN), lambda i: (0, i))],
            out_specs=[pl.BlockSpec((_GATHER_WIN, _GATHER_DIM),
                                    lambda i: (i, 0))],
            core_axis_name=("core", "subcore"),
            dimension_semantics=(pltpu.PARALLEL,),
        )(i_hbm, o_hbm)

    return gather_kernel(embed_pad, ind_flat)


def _finish_kernel(zq_ref, ze_ref, out_ref, loss_ref):
    zq = jnp.transpose(zq_ref[0, :, :EMBED_DIM], (1, 0))
    ze = ze_ref[0]
    diff = zq - ze
    out_ref[0] = ze + diff

    @pl.when(pl.program_id(0) == 0)
    def _():
        loss_ref[0, 0] = 0.0

    loss_ref[0, 0] += jnp.sum(diff * diff)


def _finish(zq_rows3, ze_t):
    return pl.pallas_call(
        _finish_kernel,
        grid=(B,),
        in_specs=[
            pl.BlockSpec((1, HW, _GATHER_DIM), lambda b: (b, 0, 0)),
            pl.BlockSpec((1, EMBED_DIM, HW), lambda b: (b, 0, 0)),
        ],
        out_specs=[
            pl.BlockSpec((1, EMBED_DIM, HW), lambda b: (b, 0, 0)),
            pl.BlockSpec(memory_space=pltpu.SMEM),
        ],
        out_shape=[
            jax.ShapeDtypeStruct((B, EMBED_DIM, HW), jnp.float32),
            jax.ShapeDtypeStruct((1, 1), jnp.float32),
        ],
    )(zq_rows3, ze_t)


def kernel(z, wait_to_init, proj_w, proj_b, embed):
    z3 = z.reshape(B, C, HW)
    proj_b2 = proj_b.reshape(EMBED_DIM, 1)
    ze_t, ind3 = _project_and_argmin(z3, proj_w, proj_b2, embed)
    ind_flat = ind3.reshape(1, _N_IDX)
    embed_pad = jnp.pad(embed, ((0, 0), (0, _GATHER_DIM - EMBED_DIM)))
    zq_rows = _sc_gather(embed_pad, ind_flat)
    zq_out, loss_sum = _finish(zq_rows.reshape(B, HW, _GATHER_DIM), ze_t)
    m = loss_sum[0, 0] / jnp.float32(_N_IDX * EMBED_DIM)
    latent_loss = (0.25 * m + m) * 10.0
    z_q = zq_out.reshape(B, EMBED_DIM, H, W)
    ind = ind3.reshape(B, H, W)
    return (z_q, latent_loss, ind)


# speedup vs baseline: 1.4261x; 1.1119x over previous
import jax
import jax.numpy as jnp
from jax.experimental import pallas as pl
from jax.experimental.pallas import tpu as pltpu
from jax.experimental.pallas import tpu_sc as plsc

B, C, H, W = 32, 384, 32, 32
HW = H * W
N_EMBED, EMBED_DIM = 8192, 64
CODE_CHUNK = 1024
N_CHUNKS = N_EMBED // CODE_CHUNK
_PREC = jax.lax.Precision.DEFAULT


def _argmin_kernel(z_ref, w_ref, b_ref, emb_ref, ze_ref, ind_ref, pad_ref):
    @pl.when(pl.program_id(0) == 0)
    def _():
        pad_ref[:, :EMBED_DIM] = emb_ref[...]
        pad_ref[:, EMBED_DIM:] = jnp.zeros((N_EMBED, _GATHER_DIM - EMBED_DIM),
                                           jnp.float32)

    x = z_ref[0]
    w = w_ref[...]
    ze = jnp.dot(w, x, preferred_element_type=jnp.float32,
                 precision=_PREC) + b_ref[...]
    ze_ref[0] = ze
    xn = jnp.sum(ze * ze, axis=0)
    best_val = jnp.full((HW,), jnp.inf, dtype=jnp.float32)
    best_idx = jnp.zeros((HW,), dtype=jnp.float32)
    iota_f = jax.lax.broadcasted_iota(
        jnp.int32, (CODE_CHUNK, HW), 0).astype(jnp.float32)
    for c in range(N_CHUNKS):
        e = emb_ref[pl.ds(c * CODE_CHUNK, CODE_CHUNK), :]
        en = jnp.sum(e * e, axis=1)
        s2 = jnp.dot(-2.0 * e, ze, preferred_element_type=jnp.float32,
                     precision=_PREC)
        dmat = (xn[None, :] + s2) + en[:, None]
        m = jnp.min(dmat, axis=0)
        cand = jnp.where(dmat == m[None, :], iota_f, jnp.float32(2**24))
        ci = jnp.min(cand, axis=0) + jnp.float32(c * CODE_CHUNK)
        upd = m < best_val
        best_val = jnp.where(upd, m, best_val)
        best_idx = jnp.where(upd, ci, best_idx)
    ind_ref[0, 0] = best_idx.astype(jnp.int32)


def _project_and_argmin(z3, proj_w, proj_b2, embed):
    return pl.pallas_call(
        _argmin_kernel,
        grid=(B,),
        in_specs=[
            pl.BlockSpec((1, C, HW), lambda b: (b, 0, 0)),
            pl.BlockSpec((EMBED_DIM, C), lambda b: (0, 0)),
            pl.BlockSpec((EMBED_DIM, 1), lambda b: (0, 0)),
            pl.BlockSpec((N_EMBED, EMBED_DIM), lambda b: (0, 0)),
        ],
        out_specs=[
            pl.BlockSpec((1, EMBED_DIM, HW), lambda b: (b, 0, 0)),
            pl.BlockSpec((1, 1, HW), lambda b: (b, 0, 0)),
            pl.BlockSpec((N_EMBED, _GATHER_DIM), lambda b: (0, 0)),
        ],
        out_shape=[
            jax.ShapeDtypeStruct((B, EMBED_DIM, HW), jnp.float32),
            jax.ShapeDtypeStruct((B, 1, HW), jnp.int32),
            jax.ShapeDtypeStruct((N_EMBED, _GATHER_DIM), jnp.float32),
        ],
        compiler_params=pltpu.CompilerParams(
            dimension_semantics=("parallel",)),
    )(z3, proj_w, proj_b2, embed)


_GATHER_WIN = 256
_GATHER_DIM = 128
_N_IDX = B * HW


def _sc_gather(embed_pad, ind_flat):
    mesh = plsc.VectorSubcoreMesh(core_axis_name="core",
                                  subcore_axis_name="subcore")

    @pl.kernel(out_type=jax.ShapeDtypeStruct((_N_IDX, _GATHER_DIM),
                                             jnp.float32),
               mesh=mesh)
    def gather_kernel(x_hbm, i_hbm, o_hbm):
        def body(i_vmem, o_vmem):
            pltpu.sync_copy(x_hbm.at[i_vmem.at[0]], o_vmem)

        pltpu.emit_pipeline(
            body,
            grid=(_N_IDX // _GATHER_WIN,),
            in_specs=[pl.BlockSpec((1, _GATHER_WIN), lambda i: (0, i))],
            out_specs=[pl.BlockSpec((_GATHER_WIN, _GATHER_DIM),
                                    lambda i: (i, 0))],
            core_axis_name=("core", "subcore"),
            dimension_semantics=(pltpu.PARALLEL,),
        )(i_hbm, o_hbm)

    return gather_kernel(embed_pad, ind_flat)


def _finish_kernel(zq_ref, ze_ref, out_ref, loss_ref):
    zq = jnp.transpose(zq_ref[0, :, :EMBED_DIM], (1, 0))
    ze = ze_ref[0]
    diff = zq - ze
    out_ref[0] = ze + diff

    @pl.when(pl.program_id(0) == 0)
    def _():
        loss_ref[0, 0] = 0.0

    loss_ref[0, 0] += jnp.sum(diff * diff)

    @pl.when(pl.program_id(0) == B - 1)
    def _():
        m = loss_ref[0, 0] / jnp.float32(_N_IDX * EMBED_DIM)
        loss_ref[0, 0] = (0.25 * m + m) * 10.0


def _finish(zq_rows3, ze_t):
    return pl.pallas_call(
        _finish_kernel,
        grid=(B,),
        in_specs=[
            pl.BlockSpec((1, HW, _GATHER_DIM), lambda b: (b, 0, 0)),
            pl.BlockSpec((1, EMBED_DIM, HW), lambda b: (b, 0, 0)),
        ],
        out_specs=[
            pl.BlockSpec((1, EMBED_DIM, HW), lambda b: (b, 0, 0)),
            pl.BlockSpec(memory_space=pltpu.SMEM),
        ],
        out_shape=[
            jax.ShapeDtypeStruct((B, EMBED_DIM, HW), jnp.float32),
            jax.ShapeDtypeStruct((1, 1), jnp.float32),
        ],
    )(zq_rows3, ze_t)


def kernel(z, wait_to_init, proj_w, proj_b, embed):
    z3 = z.reshape(B, C, HW)
    proj_b2 = proj_b.reshape(EMBED_DIM, 1)
    ze_t, ind3, embed_pad = _project_and_argmin(z3, proj_w, proj_b2, embed)
    ind_flat = ind3.reshape(1, _N_IDX)
    zq_rows = _sc_gather(embed_pad, ind_flat)
    zq_out, loss_sum = _finish(zq_rows.reshape(B, HW, _GATHER_DIM), ze_t)
    latent_loss = loss_sum[0, 0]
    z_q = zq_out.reshape(B, EMBED_DIM, H, W)
    ind = ind3.reshape(B, H, W)
    return (z_q, latent_loss, ind)


# speedup vs baseline: 1.8189x; 1.2754x over previous
import jax
import jax.numpy as jnp
from jax.experimental import pallas as pl
from jax.experimental.pallas import tpu as pltpu
from jax.experimental.pallas import tpu_sc as plsc

B, C, H, W = 32, 384, 32, 32
HW = H * W
N_EMBED, EMBED_DIM = 8192, 64
CODE_CHUNK = 1024
N_CHUNKS = N_EMBED // CODE_CHUNK
_PREC = jax.lax.Precision.DEFAULT


def _argmin_kernel(z_ref, w_ref, b_ref, emb_ref, ze_ref, ind_ref, pad_ref):
    @pl.when(pl.program_id(0) == 0)
    def _():
        pad_ref[:, :EMBED_DIM] = emb_ref[...]
        pad_ref[:, EMBED_DIM:] = jnp.zeros((N_EMBED, _GATHER_DIM - EMBED_DIM),
                                           jnp.float32)

    x = z_ref[0]
    w = w_ref[...]
    ze = jnp.dot(w, x, preferred_element_type=jnp.float32,
                 precision=_PREC) + b_ref[...]
    ze_ref[0] = ze
    xn = jnp.sum(ze * ze, axis=0)
    acc_v = jnp.full((8, HW), jnp.inf, dtype=jnp.float32)
    acc_r = jnp.zeros((8, HW), dtype=jnp.float32)
    RB = CODE_CHUNK // 8
    for c in range(N_CHUNKS):
        e = emb_ref[pl.ds(c * CODE_CHUNK, CODE_CHUNK), :]
        en = jnp.sum(e * e, axis=1)
        s2 = jnp.dot(-2.0 * e, ze, preferred_element_type=jnp.float32,
                     precision=_PREC)
        dmat = (xn[None, :] + s2) + en[:, None]
        d3 = dmat.reshape(RB, 8, HW)
        for r in range(RB):
            blk = d3[r]
            lt = blk < acc_v
            acc_v = jnp.where(lt, blk, acc_v)
            acc_r = jnp.where(lt, jnp.float32(c * RB + r), acc_r)
    sub_iota = jax.lax.broadcasted_iota(jnp.int32, (8, HW), 0).astype(
        jnp.float32)
    code = acc_r * 8.0 + sub_iota
    mv = jnp.min(acc_v, axis=0)
    cand = jnp.where(acc_v == mv[None, :], code, jnp.float32(2**24))
    ind_ref[0, 0] = jnp.min(cand, axis=0).astype(jnp.int32)


def _project_and_argmin(z3, proj_w, proj_b2, embed):
    return pl.pallas_call(
        _argmin_kernel,
        grid=(B,),
        in_specs=[
            pl.BlockSpec((1, C, HW), lambda b: (b, 0, 0)),
            pl.BlockSpec((EMBED_DIM, C), lambda b: (0, 0)),
            pl.BlockSpec((EMBED_DIM, 1), lambda b: (0, 0)),
            pl.BlockSpec((N_EMBED, EMBED_DIM), lambda b: (0, 0)),
        ],
        out_specs=[
            pl.BlockSpec((1, EMBED_DIM, HW), lambda b: (b, 0, 0)),
            pl.BlockSpec((1, 1, HW), lambda b: (b, 0, 0)),
            pl.BlockSpec((N_EMBED, _GATHER_DIM), lambda b: (0, 0)),
        ],
        out_shape=[
            jax.ShapeDtypeStruct((B, EMBED_DIM, HW), jnp.float32),
            jax.ShapeDtypeStruct((B, 1, HW), jnp.int32),
            jax.ShapeDtypeStruct((N_EMBED, _GATHER_DIM), jnp.float32),
        ],
        compiler_params=pltpu.CompilerParams(
            dimension_semantics=("parallel",)),
    )(z3, proj_w, proj_b2, embed)


_GATHER_WIN = 256
_GATHER_DIM = 128
_N_IDX = B * HW


def _sc_gather(embed_pad, ind_flat):
    mesh = plsc.VectorSubcoreMesh(core_axis_name="core",
                                  subcore_axis_name="subcore")

    @pl.kernel(out_type=jax.ShapeDtypeStruct((_N_IDX, _GATHER_DIM),
                                             jnp.float32),
               mesh=mesh)
    def gather_kernel(x_hbm, i_hbm, o_hbm):
        def body(i_vmem, o_vmem):
            pltpu.sync_copy(x_hbm.at[i_vmem.at[0]], o_vmem)

        pltpu.emit_pipeline(
            body,
            grid=(_N_IDX // _GATHER_WIN,),
            in_specs=[pl.BlockSpec((1, _GATHER_WIN), lambda i: (0, i))],
            out_specs=[pl.BlockSpec((_GATHER_WIN, _GATHER_DIM),
                                    lambda i: (i, 0))],
            core_axis_name=("core", "subcore"),
            dimension_semantics=(pltpu.PARALLEL,),
        )(i_hbm, o_hbm)

    return gather_kernel(embed_pad, ind_flat)


def _finish_kernel(zq_ref, ze_ref, out_ref, loss_ref):
    zq = jnp.transpose(zq_ref[0, :, :EMBED_DIM], (1, 0))
    ze = ze_ref[0]
    diff = zq - ze
    out_ref[0] = ze + diff

    @pl.when(pl.program_id(0) == 0)
    def _():
        loss_ref[0, 0] = 0.0

    loss_ref[0, 0] += jnp.sum(diff * diff)

    @pl.when(pl.program_id(0) == B - 1)
    def _():
        m = loss_ref[0, 0] / jnp.float32(_N_IDX * EMBED_DIM)
        loss_ref[0, 0] = (0.25 * m + m) * 10.0


def _finish(zq_rows3, ze_t):
    return pl.pallas_call(
        _finish_kernel,
        grid=(B,),
        in_specs=[
            pl.BlockSpec((1, HW, _GATHER_DIM), lambda b: (b, 0, 0)),
            pl.BlockSpec((1, EMBED_DIM, HW), lambda b: (b, 0, 0)),
        ],
        out_specs=[
            pl.BlockSpec((1, EMBED_DIM, HW), lambda b: (b, 0, 0)),
            pl.BlockSpec(memory_space=pltpu.SMEM),
        ],
        out_shape=[
            jax.ShapeDtypeStruct((B, EMBED_DIM, HW), jnp.float32),
            jax.ShapeDtypeStruct((1, 1), jnp.float32),
        ],
    )(zq_rows3, ze_t)


def kernel(z, wait_to_init, proj_w, proj_b, embed):
    z3 = z.reshape(B, C, HW)
    proj_b2 = proj_b.reshape(EMBED_DIM, 1)
    ze_t, ind3, embed_pad = _project_and_argmin(z3, proj_w, proj_b2, embed)
    ind_flat = ind3.reshape(1, _N_IDX)
    zq_rows = _sc_gather(embed_pad, ind_flat)
    zq_out, loss_sum = _finish(zq_rows.reshape(B, HW, _GATHER_DIM), ze_t)
    latent_loss = loss_sum[0, 0]
    z_q = zq_out.reshape(B, EMBED_DIM, H, W)
    ind = ind3.reshape(B, H, W)
    return (z_q, latent_loss, ind)


# speedup vs baseline: 1.8818x; 1.0346x over previous
import jax
import jax.numpy as jnp
from jax.experimental import pallas as pl
from jax.experimental.pallas import tpu as pltpu
from jax.experimental.pallas import tpu_sc as plsc

B, C, H, W = 32, 384, 32, 32
HW = H * W
N_EMBED, EMBED_DIM = 8192, 64
CODE_CHUNK = 1024
N_CHUNKS = N_EMBED // CODE_CHUNK
_PREC = jax.lax.Precision.DEFAULT


def _argmin_kernel(z_ref, w_ref, b_ref, emb_ref, ze_ref, ind_ref, pad_ref,
                   e2_ref, en_ref):
    @pl.when(pl.program_id(0) == 0)
    def _():
        pad_ref[:, :EMBED_DIM] = emb_ref[...]
        pad_ref[:, EMBED_DIM:] = jnp.zeros((N_EMBED, _GATHER_DIM - EMBED_DIM),
                                           jnp.float32)
        for c in range(N_CHUNKS):
            sl = pl.ds(c * CODE_CHUNK, CODE_CHUNK)
            e = emb_ref[sl, :]
            e2_ref[sl, :] = -2.0 * e
            en_ref[sl, :] = jnp.sum(e * e, axis=1)[:, None]

    x = z_ref[0]
    w = w_ref[...]
    ze = jnp.dot(w, x, preferred_element_type=jnp.float32,
                 precision=_PREC) + b_ref[...]
    ze_ref[0] = ze
    xn = jnp.sum(ze * ze, axis=0)
    acc_v = jnp.full((8, HW), jnp.inf, dtype=jnp.float32)
    acc_r = jnp.zeros((8, HW), dtype=jnp.float32)
    RB = CODE_CHUNK // 8
    for c in range(N_CHUNKS):
        sl = pl.ds(c * CODE_CHUNK, CODE_CHUNK)
        s2 = jnp.dot(e2_ref[sl, :], ze, preferred_element_type=jnp.float32,
                     precision=_PREC)
        dmat = (xn[None, :] + s2) + en_ref[sl, :]
        d3 = dmat.reshape(RB, 8, HW)
        for r in range(RB):
            blk = d3[r]
            lt = blk < acc_v
            acc_v = jnp.where(lt, blk, acc_v)
            acc_r = jnp.where(lt, jnp.float32(c * RB + r), acc_r)
    sub_iota = jax.lax.broadcasted_iota(jnp.int32, (8, HW), 0).astype(
        jnp.float32)
    code = acc_r * 8.0 + sub_iota
    mv = jnp.min(acc_v, axis=0)
    cand = jnp.where(acc_v == mv[None, :], code, jnp.float32(2**24))
    ind_ref[0, 0] = jnp.min(cand, axis=0).astype(jnp.int32)


def _project_and_argmin(z3, proj_w, proj_b2, embed):
    return pl.pallas_call(
        _argmin_kernel,
        grid=(B,),
        in_specs=[
            pl.BlockSpec((1, C, HW), lambda b: (b, 0, 0)),
            pl.BlockSpec((EMBED_DIM, C), lambda b: (0, 0)),
            pl.BlockSpec((EMBED_DIM, 1), lambda b: (0, 0)),
            pl.BlockSpec((N_EMBED, EMBED_DIM), lambda b: (0, 0)),
        ],
        out_specs=[
            pl.BlockSpec((1, EMBED_DIM, HW), lambda b: (b, 0, 0)),
            pl.BlockSpec((1, 1, HW), lambda b: (b, 0, 0)),
            pl.BlockSpec((N_EMBED, _GATHER_DIM), lambda b: (0, 0)),
        ],
        out_shape=[
            jax.ShapeDtypeStruct((B, EMBED_DIM, HW), jnp.float32),
            jax.ShapeDtypeStruct((B, 1, HW), jnp.int32),
            jax.ShapeDtypeStruct((N_EMBED, _GATHER_DIM), jnp.float32),
        ],
        scratch_shapes=[
            pltpu.VMEM((N_EMBED, EMBED_DIM), jnp.float32),
            pltpu.VMEM((N_EMBED, 1), jnp.float32),
        ],
        compiler_params=pltpu.CompilerParams(
            dimension_semantics=("arbitrary",)),
    )(z3, proj_w, proj_b2, embed)


_GATHER_WIN = 256
_GATHER_DIM = 128
_N_IDX = B * HW


def _sc_gather(embed_pad, ind_flat):
    mesh = plsc.VectorSubcoreMesh(core_axis_name="core",
                                  subcore_axis_name="subcore")

    @pl.kernel(out_type=jax.ShapeDtypeStruct((_N_IDX, _GATHER_DIM),
                                             jnp.float32),
               mesh=mesh)
    def gather_kernel(x_hbm, i_hbm, o_hbm):
        def body(i_vmem, o_vmem):
            pltpu.sync_copy(x_hbm.at[i_vmem.at[0]], o_vmem)

        pltpu.emit_pipeline(
            body,
            grid=(_N_IDX // _GATHER_WIN,),
            in_specs=[pl.BlockSpec((1, _GATHER_WIN), lambda i: (0, i))],
            out_specs=[pl.BlockSpec((_GATHER_WIN, _GATHER_DIM),
                                    lambda i: (i, 0))],
            core_axis_name=("core", "subcore"),
            dimension_semantics=(pltpu.PARALLEL,),
        )(i_hbm, o_hbm)

    return gather_kernel(embed_pad, ind_flat)


def _finish_kernel(zq_ref, ze_ref, out_ref, loss_ref):
    zq = jnp.transpose(zq_ref[0, :, :EMBED_DIM], (1, 0))
    ze = ze_ref[0]
    diff = zq - ze
    out_ref[0] = ze + diff

    @pl.when(pl.program_id(0) == 0)
    def _():
        loss_ref[0, 0] = 0.0

    loss_ref[0, 0] += jnp.sum(diff * diff)

    @pl.when(pl.program_id(0) == B - 1)
    def _():
        m = loss_ref[0, 0] / jnp.float32(_N_IDX * EMBED_DIM)
        loss_ref[0, 0] = (0.25 * m + m) * 10.0


def _finish(zq_rows3, ze_t):
    return pl.pallas_call(
        _finish_kernel,
        grid=(B,),
        in_specs=[
            pl.BlockSpec((1, HW, _GATHER_DIM), lambda b: (b, 0, 0)),
            pl.BlockSpec((1, EMBED_DIM, HW), lambda b: (b, 0, 0)),
        ],
        out_specs=[
            pl.BlockSpec((1, EMBED_DIM, HW), lambda b: (b, 0, 0)),
            pl.BlockSpec(memory_space=pltpu.SMEM),
        ],
        out_shape=[
            jax.ShapeDtypeStruct((B, EMBED_DIM, HW), jnp.float32),
            jax.ShapeDtypeStruct((1, 1), jnp.float32),
        ],
    )(zq_rows3, ze_t)


def kernel(z, wait_to_init, proj_w, proj_b, embed):
    z3 = z.reshape(B, C, HW)
    proj_b2 = proj_b.reshape(EMBED_DIM, 1)
    ze_t, ind3, embed_pad = _project_and_argmin(z3, proj_w, proj_b2, embed)
    ind_flat = ind3.reshape(1, _N_IDX)
    zq_rows = _sc_gather(embed_pad, ind_flat)
    zq_out, loss_sum = _finish(zq_rows.reshape(B, HW, _GATHER_DIM), ze_t)
    latent_loss = loss_sum[0, 0]
    z_q = zq_out.reshape(B, EMBED_DIM, H, W)
    ind = ind3.reshape(B, H, W)
    return (z_q, latent_loss, ind)
